# baseline (speedup 1.0000x reference)
import jax
import jax.numpy as jnp
from jax import lax
from jax.experimental import pallas as pl
from jax.experimental.pallas import tpu as pltpu

_B = 32
_A = 102
_H = 224
_S = _H * _H
_R = 16
_NCH = _B * _A // _R
_NBUF = 6


def _pool_body(x_hbm, o_ref, *scratch):
    bufs = scratch[:_NBUF]
    sems = scratch[_NBUF:]

    def cp(j, slot):
        return pltpu.make_async_copy(
            x_hbm.at[pl.ds(j * _R, _R)], bufs[slot], sems[slot])

    for s in range(_NBUF):
        cp(s, s).start()

    def outer(o, carry):
        base = o * _NBUF
        for b in range(_NBUF):
            i = base + b
            cp(i, b).wait()
            partial = jnp.sum(bufs[b][...], axis=1)
            s = jnp.sum(partial, axis=1, keepdims=True) * (1.0 / _S)
            o_ref[pl.ds(i * _R, _R), :] = s
            nxt = i + _NBUF

            @pl.when(nxt < _NCH)
            def _():
                cp(nxt, b).start()
        return carry

    lax.fori_loop(0, _NCH // _NBUF, outer, 0)


def _gate_body(x_ref, w_ref, o_ref):
    x = x_ref[...]
    w = w_ref[...]
    vote = x * w
    absv = jnp.abs(vote)
    dense = jnp.sum(vote, axis=1)
    iota = lax.broadcasted_iota(jnp.int32, (_B, _A), 1)
    acc = jnp.zeros((_B,), jnp.float32)
    outs = []
    for _ in range(8):
        m = jnp.max(absv, axis=1, keepdims=True)
        ismax = absv == m
        first = jnp.min(jnp.where(ismax, iota, _A), axis=1, keepdims=True)
        onehot = iota == first
        acc = acc + jnp.sum(jnp.where(onehot, vote, 0.0), axis=1)
        outs.append(acc + 0.5)
        absv = jnp.where(onehot, -1.0, absv)
    outs.append(dense + 0.5)
    o_ref[...] = jnp.stack(outs, axis=0)


def kernel(maps, W2):
    n = _B * _A
    maps3 = maps.reshape(n, _H, _H)
    sums = pl.pallas_call(
        _pool_body,
        in_specs=[pl.BlockSpec(memory_space=pl.ANY)],
        out_specs=pl.BlockSpec(memory_space=pltpu.MemorySpace.VMEM),
        out_shape=jax.ShapeDtypeStruct((n, 1), jnp.float32),
        scratch_shapes=(
            [pltpu.VMEM((_R, _H, _H), jnp.float32)] * _NBUF
            + [pltpu.SemaphoreType.DMA] * _NBUF
        ),
    )(maps3)
    x_sun = sums.reshape(_B, _A)

    son = pl.pallas_call(
        _gate_body,
        out_shape=jax.ShapeDtypeStruct((9, _B), jnp.float32),
    )(x_sun, W2)
    x_son = son.reshape(9, _B, 1)
    return (x_sun, x_son, maps)


# speedup vs baseline: 1.4355x; 1.4355x over previous
import jax
import jax.numpy as jnp
from jax import lax
from jax.experimental import pallas as pl
from jax.experimental.pallas import tpu as pltpu

_B = 32
_A = 102
_H = 224
_S = _H * _H
_R = 16
_NCH = _B * _A // _R
_NBUF = 6


def _pool_body(x_hbm, o_ref, y_hbm, *scratch):
    bufs = scratch[:_NBUF]
    isems = scratch[_NBUF:2 * _NBUF]
    osems = scratch[2 * _NBUF:]

    def cp_in(j, slot):
        return pltpu.make_async_copy(
            x_hbm.at[pl.ds(j * _R, _R)], bufs[slot], isems[slot])

    def cp_out(j, slot):
        return pltpu.make_async_copy(
            bufs[slot], y_hbm.at[pl.ds(j * _R, _R)], osems[slot])

    for s in range(_NBUF):
        cp_in(s, s).start()

    def outer(o, carry):
        base = o * _NBUF
        for b in range(_NBUF):
            i = base + b
            cp_in(i, b).wait()
            cp_out(i, b).start()
            partial = jnp.sum(bufs[b][...], axis=1)
            s = jnp.sum(partial, axis=1, keepdims=True) * (1.0 / _S)
            o_ref[pl.ds(i * _R, _R), :] = s
            nxt = i + _NBUF

            @pl.when(nxt < _NCH)
            def _():
                cp_out(i, b).wait()
                cp_in(nxt, b).start()
        return carry

    lax.fori_loop(0, _NCH // _NBUF, outer, 0)

    for s in range(_NBUF):
        cp_out(_NCH - _NBUF + s, s).wait()


def _gate_body(x_ref, w_ref, o_ref):
    x = x_ref[...]
    w = w_ref[...]
    vote = x * w
    absv = jnp.abs(vote)
    dense = jnp.sum(vote, axis=1)
    iota = lax.broadcasted_iota(jnp.int32, (_B, _A), 1)
    acc = jnp.zeros((_B,), jnp.float32)
    outs = []
    for _ in range(8):
        m = jnp.max(absv, axis=1, keepdims=True)
        ismax = absv == m
        first = jnp.min(jnp.where(ismax, iota, _A), axis=1, keepdims=True)
        onehot = iota == first
        acc = acc + jnp.sum(jnp.where(onehot, vote, 0.0), axis=1)
        outs.append(acc + 0.5)
        absv = jnp.where(onehot, -1.0, absv)
    outs.append(dense + 0.5)
    o_ref[...] = jnp.stack(outs, axis=0)


def kernel(maps, W2):
    n = _B * _A
    maps3 = maps.reshape(n, _H, _H)
    sums, maps_out = pl.pallas_call(
        _pool_body,
        in_specs=[pl.BlockSpec(memory_space=pl.ANY)],
        out_specs=[
            pl.BlockSpec(memory_space=pltpu.MemorySpace.VMEM),
            pl.BlockSpec(memory_space=pl.ANY),
        ],
        out_shape=[
            jax.ShapeDtypeStruct((n, 1), jnp.float32),
            jax.ShapeDtypeStruct((n, _H, _H), jnp.float32),
        ],
        scratch_shapes=(
            [pltpu.VMEM((_R, _H, _H), jnp.float32)] * _NBUF
            + [pltpu.SemaphoreType.DMA] * (2 * _NBUF)
        ),
    )(maps3)
    x_sun = sums.reshape(_B, _A)

    son = pl.pallas_call(
        _gate_body,
        out_shape=jax.ShapeDtypeStruct((9, _B), jnp.float32),
    )(x_sun, W2)
    x_son = son.reshape(9, _B, 1)
    return (x_sun, x_son, maps_out.reshape(_B, _A, _H, _H))


# speedup vs baseline: 1.4557x; 1.0141x over previous
import jax
import jax.numpy as jnp
from jax import lax
from jax.experimental import pallas as pl
from jax.experimental.pallas import tpu as pltpu

_B = 32
_A = 102
_H = 224
_S = _H * _H
_R = 32
_NCH = _B * _A // _R
_NBUF = 6


def _pool_body(x_hbm, o_ref, y_hbm, *scratch):
    bufs = scratch[:_NBUF]
    isems = scratch[_NBUF:2 * _NBUF]
    osems = scratch[2 * _NBUF:]

    def cp_in(j, slot):
        return pltpu.make_async_copy(
            x_hbm.at[pl.ds(j * _R, _R)], bufs[slot], isems[slot])

    def cp_out(j, slot):
        return pltpu.make_async_copy(
            bufs[slot], y_hbm.at[pl.ds(j * _R, _R)], osems[slot])

    for s in range(_NBUF):
        cp_in(s, s).start()

    def outer(o, carry):
        base = o * _NBUF
        for b in range(_NBUF):
            i = base + b
            cp_in(i, b).wait()
            cp_out(i, b).start()
            partial = jnp.sum(bufs[b][...], axis=1)
            s = jnp.sum(partial, axis=1, keepdims=True) * (1.0 / _S)
            o_ref[pl.ds(i * _R, _R), :] = s
            nxt = i + _NBUF

            @pl.when(nxt < _NCH)
            def _():
                cp_out(i, b).wait()
                cp_in(nxt, b).start()
        return carry

    lax.fori_loop(0, _NCH // _NBUF, outer, 0)

    for s in range(_NBUF):
        cp_out(_NCH - _NBUF + s, s).wait()


def _gate_body(x_ref, w_ref, o_ref):
    x = x_ref[...]
    w = w_ref[...]
    vote = x * w
    absv = jnp.abs(vote)
    dense = jnp.sum(vote, axis=1)
    iota = lax.broadcasted_iota(jnp.int32, (_B, _A), 1)
    acc = jnp.zeros((_B,), jnp.float32)
    outs = []
    for _ in range(8):
        m = jnp.max(absv, axis=1, keepdims=True)
        ismax = absv == m
        first = jnp.min(jnp.where(ismax, iota, _A), axis=1, keepdims=True)
        onehot = iota == first
        acc = acc + jnp.sum(jnp.where(onehot, vote, 0.0), axis=1)
        outs.append(acc + 0.5)
        absv = jnp.where(onehot, -1.0, absv)
    outs.append(dense + 0.5)
    o_ref[...] = jnp.stack(outs, axis=0)


def kernel(maps, W2):
    n = _B * _A
    maps3 = maps.reshape(n, _H, _H)
    sums, maps_out = pl.pallas_call(
        _pool_body,
        in_specs=[pl.BlockSpec(memory_space=pl.ANY)],
        out_specs=[
            pl.BlockSpec(memory_space=pltpu.MemorySpace.VMEM),
            pl.BlockSpec(memory_space=pl.ANY),
        ],
        out_shape=[
            jax.ShapeDtypeStruct((n, 1), jnp.float32),
            jax.ShapeDtypeStruct((n, _H, _H), jnp.float32),
        ],
        scratch_shapes=(
            [pltpu.VMEM((_R, _H, _H), jnp.float32)] * _NBUF
            + [pltpu.SemaphoreType.DMA] * (2 * _NBUF)
        ),
    )(maps3)
    x_sun = sums.reshape(_B, _A)

    son = pl.pallas_call(
        _gate_body,
        out_shape=jax.ShapeDtypeStruct((9, _B), jnp.float32),
    )(x_sun, W2)
    x_son = son.reshape(9, _B, 1)
    return (x_sun, x_son, maps_out.reshape(_B, _A, _H, _H))


# speedup vs baseline: 1.4641x; 1.0058x over previous
import jax
import jax.numpy as jnp
from jax import lax
from jax.experimental import pallas as pl
from jax.experimental.pallas import tpu as pltpu

_B = 32
_A = 102
_H = 224
_S = _H * _H
_R = 48
_NCH = _B * _A // _R
_NBUF = 4


def _pool_body(x_hbm, o_ref, y_hbm, *scratch):
    bufs = scratch[:_NBUF]
    isems = scratch[_NBUF:2 * _NBUF]
    osems = scratch[2 * _NBUF:]

    def cp_in(j, slot):
        return pltpu.make_async_copy(
            x_hbm.at[pl.ds(j * _R, _R)], bufs[slot], isems[slot])

    def cp_out(j, slot):
        return pltpu.make_async_copy(
            bufs[slot], y_hbm.at[pl.ds(j * _R, _R)], osems[slot])

    for s in range(_NBUF):
        cp_in(s, s).start()

    def outer(o, carry):
        base = o * _NBUF
        for b in range(_NBUF):
            i = base + b
            cp_in(i, b).wait()
            cp_out(i, b).start()
            partial = jnp.sum(bufs[b][...], axis=1)
            s = jnp.sum(partial, axis=1, keepdims=True) * (1.0 / _S)
            o_ref[pl.ds(i * _R, _R), :] = s
            nxt = i + _NBUF

            @pl.when(nxt < _NCH)
            def _():
                cp_out(i, b).wait()
                cp_in(nxt, b).start()
        return carry

    lax.fori_loop(0, _NCH // _NBUF, outer, 0)

    for s in range(_NBUF):
        cp_out(_NCH - _NBUF + s, s).wait()


def _gate_body(x_ref, w_ref, o_ref):
    x = x_ref[...]
    w = w_ref[...]
    vote = x * w
    absv = jnp.abs(vote)
    dense = jnp.sum(vote, axis=1)
    iota = lax.broadcasted_iota(jnp.int32, (_B, _A), 1)
    acc = jnp.zeros((_B,), jnp.float32)
    outs = []
    for _ in range(8):
        m = jnp.max(absv, axis=1, keepdims=True)
        ismax = absv == m
        first = jnp.min(jnp.where(ismax, iota, _A), axis=1, keepdims=True)
        onehot = iota == first
        acc = acc + jnp.sum(jnp.where(onehot, vote, 0.0), axis=1)
        outs.append(acc + 0.5)
        absv = jnp.where(onehot, -1.0, absv)
    outs.append(dense + 0.5)
    o_ref[...] = jnp.stack(outs, axis=0)


def kernel(maps, W2):
    n = _B * _A
    maps3 = maps.reshape(n, _H, _H)
    sums, maps_out = pl.pallas_call(
        _pool_body,
        in_specs=[pl.BlockSpec(memory_space=pl.ANY)],
        out_specs=[
            pl.BlockSpec(memory_space=pltpu.MemorySpace.VMEM),
            pl.BlockSpec(memory_space=pl.ANY),
        ],
        out_shape=[
            jax.ShapeDtypeStruct((n, 1), jnp.float32),
            jax.ShapeDtypeStruct((n, _H, _H), jnp.float32),
        ],
        scratch_shapes=(
            [pltpu.VMEM((_R, _H, _H), jnp.float32)] * _NBUF
            + [pltpu.SemaphoreType.DMA] * (2 * _NBUF)
        ),
    )(maps3)
    x_sun = sums.reshape(_B, _A)

    son = pl.pallas_call(
        _gate_body,
        out_shape=jax.ShapeDtypeStruct((9, _B), jnp.float32),
    )(x_sun, W2)
    x_son = son.reshape(9, _B, 1)
    return (x_sun, x_son, maps_out.reshape(_B, _A, _H, _H))
